# baseline (speedup 1.0000x reference)
import functools

import jax
import jax.numpy as jnp
from jax import lax
from jax.experimental import pallas as pl
from jax.experimental.pallas import tpu as pltpu
from jax.experimental.pallas import tpu_sc as plsc

NC = 2
NS = 16
L = 16
NW = NC * NS
B = 16384
BPW = B // NW
CHUNK = 128
NCHUNK = BPW // CHUNK
D = 16
G = BPW // L

NU = 1000000
NQ = 100000
BLK_U = 262144
BLK_Q = 131072
NJ_U = -(-NU // BLK_U)
NJ_Q = -(-NQ // BLK_Q)
NUP = NJ_U * BLK_U
NQP = NJ_Q * BLK_Q


def _untile_body(x_ref, o_ref):
    c = pl.program_id(1)
    o_ref[...] = x_ref[pl.ds(c, 1), :]


def _untile(table_t, v, nj, blk):
    return pl.pallas_call(
        _untile_body,
        grid=(nj, D),
        in_specs=[pl.BlockSpec((D, blk), lambda j, c: (0, j))],
        out_specs=pl.BlockSpec((1, blk), lambda j, c: (0, c * nj + j)),
        out_shape=jax.ShapeDtypeStruct((1, D * nj * blk), jnp.float32),
        compiler_params=pltpu.CompilerParams(
            dimension_semantics=("arbitrary", "arbitrary"),
            vmem_limit_bytes=100 * 1024 * 1024,
        ),
    )(table_t)


_mesh = plsc.VectorSubcoreMesh(core_axis_name="c", subcore_axis_name="s")


@functools.partial(
    pl.kernel,
    out_type=jax.ShapeDtypeStruct((B,), jnp.float32),
    mesh=_mesh,
    compiler_params=pltpu.CompilerParams(
        needs_layout_passes=False,
        use_tc_tiling_on_sc=False,
    ),
    scratch_types=[
        pltpu.VMEM((BPW,), jnp.int32),
        pltpu.VMEM((BPW,), jnp.int32),
        pltpu.VMEM((BPW * D,), jnp.float32),
        pltpu.VMEM((BPW * D,), jnp.float32),
        pltpu.VMEM((BPW,), jnp.float32),
        pltpu.VMEM((BPW,), jnp.float32),
        pltpu.SemaphoreType.DMA,
    ],
)
def _mirt_sc(uid_hbm, qid_hbm, theta_hbm, a_hbm, b_hbm, out_hbm,
             uid_v, qid_v, th_cm, a_cm, b_v, out_v, sem):
    wid = lax.axis_index("s") * NC + lax.axis_index("c")
    base = wid * BPW
    pltpu.sync_copy(uid_hbm.at[pl.ds(base, BPW)], uid_v)
    pltpu.sync_copy(qid_hbm.at[pl.ds(base, BPW)], qid_v)

    th_flat = theta_hbm.at[0]
    a_flat = a_hbm.at[0]

    copies = []
    for j in range(NCHUNK):
        sl = pl.ds(j * CHUNK, CHUNK)
        for c in range(D):
            dst = pl.ds(c * BPW + j * CHUNK, CHUNK)
            copies.append(pltpu.make_async_copy(
                th_flat.at[pl.ds(c * NUP, NUP)].at[uid_v.at[sl]],
                th_cm.at[dst], sem))
            copies.append(pltpu.make_async_copy(
                a_flat.at[pl.ds(c * NQP, NQP)].at[qid_v.at[sl]],
                a_cm.at[dst], sem))
        copies.append(pltpu.make_async_copy(
            b_hbm.at[0].at[qid_v.at[sl]], b_v.at[sl], sem))
    for cp in copies:
        cp.start()
    for cp in copies:
        cp.wait()

    def group(g, carry):
        row0 = pl.multiple_of(g * L, L)
        acc = jnp.zeros((L,), jnp.float32)
        for c in range(D):
            av = a_cm[pl.ds(c * BPW + row0, L)]
            tv = th_cm[pl.ds(c * BPW + row0, L)]
            acc = acc + tv / (1.0 + jnp.exp(-av))
        bb = b_v[pl.ds(row0, L)]
        out_v[pl.ds(row0, L)] = 1.0 / (1.0 + jnp.exp(bb - acc))
        return carry

    lax.fori_loop(0, G, group, 0)
    pltpu.sync_copy(out_v, out_hbm.at[pl.ds(base, BPW)])


def kernel(user_id, question_id, theta_w, a_w, b_w):
    th_flat = _untile(theta_w.T, NU, NJ_U, BLK_U)
    a_flat = _untile(a_w.T, NQ, NJ_Q, BLK_Q)
    return _mirt_sc(user_id.astype(jnp.int32), question_id.astype(jnp.int32),
                    th_flat, a_flat, b_w.T)


# speedup vs baseline: 1.3086x; 1.3086x over previous
import functools

import jax
import jax.numpy as jnp
from jax import lax
from jax.experimental import pallas as pl
from jax.experimental.pallas import tpu as pltpu
from jax.experimental.pallas import tpu_sc as plsc

NC = 2
NS = 16
L = 16
NW = NC * NS
B = 16384
BPW = B // NW
CHUNK = 128
NCHUNK = BPW // CHUNK
D = 16
G = BPW // L

NU = 1000000
NQ = 100000
BLK_U = 131072
BLK_Q = 131072
NJ_U = -(-NU // BLK_U)
NJ_Q = -(-NQ // BLK_Q)
NUP = NJ_U * BLK_U
NQP = NJ_Q * BLK_Q


def _untile_body(x_ref, *o_refs):
    for c in range(D):
        o_refs[c][...] = x_ref[pl.ds(c, 1), :]


def _untile(table_t, nj, blk):
    return pl.pallas_call(
        _untile_body,
        grid=(nj,),
        in_specs=[pl.BlockSpec((D, blk), lambda j: (0, j))],
        out_specs=[pl.BlockSpec((1, blk), lambda j: (0, j))] * D,
        out_shape=[jax.ShapeDtypeStruct((1, nj * blk), jnp.float32)] * D,
        compiler_params=pltpu.CompilerParams(
            dimension_semantics=("arbitrary",),
        ),
    )(table_t)


_mesh = plsc.VectorSubcoreMesh(core_axis_name="c", subcore_axis_name="s")


@functools.partial(
    pl.kernel,
    out_type=jax.ShapeDtypeStruct((B,), jnp.float32),
    mesh=_mesh,
    compiler_params=pltpu.CompilerParams(
        needs_layout_passes=False,
        use_tc_tiling_on_sc=False,
    ),
    scratch_types=[
        pltpu.VMEM((BPW,), jnp.int32),
        pltpu.VMEM((BPW,), jnp.int32),
        pltpu.VMEM((BPW * D,), jnp.float32),
        pltpu.VMEM((BPW * D,), jnp.float32),
        pltpu.VMEM((BPW,), jnp.float32),
        pltpu.VMEM((BPW,), jnp.float32),
        pltpu.SemaphoreType.DMA,
    ],
)
def _mirt_sc(uid_hbm, qid_hbm, *rest):
    th_refs = rest[0:D]
    a_refs = rest[D:2 * D]
    b_hbm = rest[2 * D]
    out_hbm = rest[2 * D + 1]
    uid_v, qid_v, th_cm, a_cm, b_v, out_v, sem = rest[2 * D + 2:]

    wid = lax.axis_index("s") * NC + lax.axis_index("c")
    base = wid * BPW
    pltpu.sync_copy(uid_hbm.at[pl.ds(base, BPW)], uid_v)
    pltpu.sync_copy(qid_hbm.at[pl.ds(base, BPW)], qid_v)

    copies = []
    for j in range(NCHUNK):
        sl = pl.ds(j * CHUNK, CHUNK)
        for c in range(D):
            dst = pl.ds(c * BPW + j * CHUNK, CHUNK)
            copies.append(pltpu.make_async_copy(
                th_refs[c].at[0].at[uid_v.at[sl]], th_cm.at[dst], sem))
            copies.append(pltpu.make_async_copy(
                a_refs[c].at[0].at[qid_v.at[sl]], a_cm.at[dst], sem))
        copies.append(pltpu.make_async_copy(
            b_hbm.at[0].at[qid_v.at[sl]], b_v.at[sl], sem))
    for cp in copies:
        cp.start()
    for cp in copies:
        cp.wait()

    def group(g, carry):
        row0 = pl.multiple_of(g * L, L)
        acc = jnp.zeros((L,), jnp.float32)
        for c in range(D):
            av = a_cm[pl.ds(c * BPW + row0, L)]
            tv = th_cm[pl.ds(c * BPW + row0, L)]
            acc = acc + tv / (1.0 + jnp.exp(-av))
        bb = b_v[pl.ds(row0, L)]
        out_v[pl.ds(row0, L)] = 1.0 / (1.0 + jnp.exp(bb - acc))
        return carry

    lax.fori_loop(0, G, group, 0)
    pltpu.sync_copy(out_v, out_hbm.at[pl.ds(base, BPW)])


def kernel(user_id, question_id, theta_w, a_w, b_w):
    th_list = _untile(theta_w.T, NJ_U, BLK_U)
    a_list = _untile(a_w.T, NJ_Q, BLK_Q)
    return _mirt_sc(user_id.astype(jnp.int32), question_id.astype(jnp.int32),
                    *th_list, *a_list, b_w.T)


# speedup vs baseline: 1.3243x; 1.0120x over previous
import functools

import jax
import jax.numpy as jnp
from jax import lax
from jax.experimental import pallas as pl
from jax.experimental.pallas import tpu as pltpu
from jax.experimental.pallas import tpu_sc as plsc

NC = 2
NS = 16
L = 16
NW = NC * NS
B = 16384
BPW = B // NW
CHUNK = 128
NCHUNK = BPW // CHUNK
D = 16
G = BPW // L

NU = 1000000
NQ = 100000
BLK_U = 131072
BLK_Q = 131072
NJ_U = -(-NU // BLK_U)
NJ_Q = -(-NQ // BLK_Q)
NUP = NJ_U * BLK_U
NQP = NJ_Q * BLK_Q


def _untile_body(th_ref, a_ref, *o_refs):
    for c in range(D):
        o_refs[c][...] = th_ref[pl.ds(c, 1), :]

    @pl.when(pl.program_id(0) == 0)
    def _():
        for c in range(D):
            o_refs[D + c][...] = a_ref[pl.ds(c, 1), :]


def _untile(theta_t, a_t):
    return pl.pallas_call(
        _untile_body,
        grid=(NJ_U,),
        in_specs=[
            pl.BlockSpec((D, BLK_U), lambda j: (0, j)),
            pl.BlockSpec((D, BLK_Q), lambda j: (0, 0)),
        ],
        out_specs=(
            [pl.BlockSpec((1, BLK_U), lambda j: (0, j))] * D
            + [pl.BlockSpec((1, BLK_Q), lambda j: (0, 0))] * D
        ),
        out_shape=(
            [jax.ShapeDtypeStruct((1, NUP), jnp.float32)] * D
            + [jax.ShapeDtypeStruct((1, NQP), jnp.float32)] * D
        ),
        compiler_params=pltpu.CompilerParams(
            dimension_semantics=("arbitrary",),
        ),
    )(theta_t, a_t)


_mesh = plsc.VectorSubcoreMesh(core_axis_name="c", subcore_axis_name="s")


@functools.partial(
    pl.kernel,
    out_type=jax.ShapeDtypeStruct((B,), jnp.float32),
    mesh=_mesh,
    compiler_params=pltpu.CompilerParams(
        needs_layout_passes=False,
        use_tc_tiling_on_sc=False,
    ),
    scratch_types=[
        pltpu.VMEM((BPW,), jnp.int32),
        pltpu.VMEM((BPW,), jnp.int32),
        pltpu.VMEM((BPW * D,), jnp.float32),
        pltpu.VMEM((BPW * D,), jnp.float32),
        pltpu.VMEM((BPW,), jnp.float32),
        pltpu.VMEM((BPW,), jnp.float32),
        pltpu.SemaphoreType.DMA,
    ],
)
def _mirt_sc(uid_hbm, qid_hbm, *rest):
    th_refs = rest[0:D]
    a_refs = rest[D:2 * D]
    b_hbm = rest[2 * D]
    out_hbm = rest[2 * D + 1]
    uid_v, qid_v, th_cm, a_cm, b_v, out_v, sem = rest[2 * D + 2:]

    wid = lax.axis_index("s") * NC + lax.axis_index("c")
    base = wid * BPW
    pltpu.sync_copy(uid_hbm.at[pl.ds(base, BPW)], uid_v)
    pltpu.sync_copy(qid_hbm.at[pl.ds(base, BPW)], qid_v)

    copies = []
    for j in range(NCHUNK):
        sl = pl.ds(j * CHUNK, CHUNK)
        for c in range(D):
            dst = pl.ds(c * BPW + j * CHUNK, CHUNK)
            copies.append(pltpu.make_async_copy(
                th_refs[c].at[0].at[uid_v.at[sl]], th_cm.at[dst], sem))
            copies.append(pltpu.make_async_copy(
                a_refs[c].at[0].at[qid_v.at[sl]], a_cm.at[dst], sem))
        copies.append(pltpu.make_async_copy(
            b_hbm.at[0].at[qid_v.at[sl]], b_v.at[sl], sem))
    for cp in copies:
        cp.start()
    for cp in copies:
        cp.wait()

    def group(g, carry):
        row0 = pl.multiple_of(g * L, L)
        acc = jnp.zeros((L,), jnp.float32)
        for c in range(D):
            av = a_cm[pl.ds(c * BPW + row0, L)]
            tv = th_cm[pl.ds(c * BPW + row0, L)]
            acc = acc + tv / (1.0 + jnp.exp(-av))
        bb = b_v[pl.ds(row0, L)]
        out_v[pl.ds(row0, L)] = 1.0 / (1.0 + jnp.exp(bb - acc))
        return carry

    lax.fori_loop(0, G, group, 0)
    pltpu.sync_copy(out_v, out_hbm.at[pl.ds(base, BPW)])


def kernel(user_id, question_id, theta_w, a_w, b_w):
    staged = _untile(theta_w.T, a_w.T)
    return _mirt_sc(user_id.astype(jnp.int32), question_id.astype(jnp.int32),
                    *staged, b_w.T)


# speedup vs baseline: 1.3309x; 1.0050x over previous
import functools

import jax
import jax.numpy as jnp
from jax import lax
from jax.experimental import pallas as pl
from jax.experimental.pallas import tpu as pltpu
from jax.experimental.pallas import tpu_sc as plsc

NC = 2
NS = 16
L = 16
NW = NC * NS
B = 16384
BPW = B // NW
CHUNK = 128
NCHUNK = BPW // CHUNK
D = 16
G = BPW // L

NU = 1000000
NQ = 100000
BLK_U = 131072
BLK_Q = 131072
NJ_U = -(-NU // BLK_U)
NJ_Q = -(-NQ // BLK_Q)
NUP = NJ_U * BLK_U
NQP = NJ_Q * BLK_Q


def _untile_body(th_ref, a_ref, *o_refs):
    for c in range(D):
        o_refs[c][...] = th_ref[pl.ds(c, 1), :]

    @pl.when(pl.program_id(0) == 0)
    def _():
        for c in range(D):
            o_refs[D + c][...] = a_ref[pl.ds(c, 1), :]


def _untile(theta_t, a_t):
    return pl.pallas_call(
        _untile_body,
        grid=(NJ_U,),
        in_specs=[
            pl.BlockSpec((D, BLK_U), lambda j: (0, j)),
            pl.BlockSpec((D, BLK_Q), lambda j: (0, 0)),
        ],
        out_specs=(
            [pl.BlockSpec((1, BLK_U), lambda j: (0, j))] * D
            + [pl.BlockSpec((1, BLK_Q), lambda j: (0, 0))] * D
        ),
        out_shape=(
            [jax.ShapeDtypeStruct((1, NUP), jnp.float32)] * D
            + [jax.ShapeDtypeStruct((1, NQP), jnp.float32)] * D
        ),
        compiler_params=pltpu.CompilerParams(
            dimension_semantics=("arbitrary",),
        ),
    )(theta_t, a_t)


_mesh = plsc.VectorSubcoreMesh(core_axis_name="c", subcore_axis_name="s")


@functools.partial(
    pl.kernel,
    out_type=jax.ShapeDtypeStruct((B,), jnp.float32),
    mesh=_mesh,
    compiler_params=pltpu.CompilerParams(
        needs_layout_passes=False,
        use_tc_tiling_on_sc=False,
    ),
    scratch_types=[
        pltpu.VMEM((BPW,), jnp.int32),
        pltpu.VMEM((BPW,), jnp.int32),
        pltpu.VMEM((BPW * D,), jnp.float32),
        pltpu.VMEM((BPW * D,), jnp.float32),
        pltpu.VMEM((BPW,), jnp.float32),
        pltpu.VMEM((BPW,), jnp.float32),
        pltpu.SemaphoreType.DMA,
    ],
)
def _mirt_sc(uid_hbm, qid_hbm, *rest):
    th_refs = rest[0:D]
    a_refs = rest[D:2 * D]
    b_hbm = rest[2 * D]
    out_hbm = rest[2 * D + 1]
    uid_v, qid_v, th_cm, a_cm, b_v, out_v, sem = rest[2 * D + 2:]

    wid = lax.axis_index("s") * NC + lax.axis_index("c")
    base = wid * BPW
    pltpu.sync_copy(uid_hbm.at[pl.ds(base, BPW)], uid_v)
    pltpu.sync_copy(qid_hbm.at[pl.ds(base, BPW)], qid_v)

    copies = []
    for c in range(D):
        dst = pl.ds(c * BPW, BPW)
        copies.append(pltpu.make_async_copy(
            th_refs[c].at[0].at[uid_v], th_cm.at[dst], sem))
        copies.append(pltpu.make_async_copy(
            a_refs[c].at[0].at[qid_v], a_cm.at[dst], sem))
    copies.append(pltpu.make_async_copy(b_hbm.at[0].at[qid_v], b_v, sem))
    for cp in copies:
        cp.start()
    for cp in copies:
        cp.wait()

    def group(g, carry):
        row0 = pl.multiple_of(g * L, L)
        acc = jnp.zeros((L,), jnp.float32)
        for c in range(D):
            av = a_cm[pl.ds(c * BPW + row0, L)]
            tv = th_cm[pl.ds(c * BPW + row0, L)]
            acc = acc + tv / (1.0 + jnp.exp(-av))
        bb = b_v[pl.ds(row0, L)]
        out_v[pl.ds(row0, L)] = 1.0 / (1.0 + jnp.exp(bb - acc))
        return carry

    lax.fori_loop(0, G, group, 0)
    pltpu.sync_copy(out_v, out_hbm.at[pl.ds(base, BPW)])


def kernel(user_id, question_id, theta_w, a_w, b_w):
    staged = _untile(theta_w.T, a_w.T)
    return _mirt_sc(user_id.astype(jnp.int32), question_id.astype(jnp.int32),
                    *staged, b_w.T)
